# baseline (speedup 1.0000x reference)
import functools

import jax
import jax.numpy as jnp
from jax import lax
from jax.experimental import pallas as pl
from jax.experimental.pallas import tpu as pltpu
from jax.experimental.pallas import tpu_sc as plsc

B, S, D = 4, 8192, 768
NC, NS = 2, 16
NW = NC * NS
POS_PER_W = S // NW
CH = 32
NCHUNK = POS_PER_W // CH
CHW = CH * D

_mesh = plsc.VectorSubcoreMesh(core_axis_name="c", subcore_axis_name="s")


@functools.partial(
    pl.kernel,
    out_type=jax.ShapeDtypeStruct((B * S * D,), jnp.float32),
    mesh=_mesh,
    scratch_types=[
        pltpu.VMEM((CHW,), jnp.float32),
        pltpu.VMEM((CHW,), jnp.float32),
    ],
)
def _sc_add(x_hbm, pos_hbm, out_hbm, pos_v, x_v):
    wid = lax.axis_index("s") * NC + lax.axis_index("c")
    pos_base = wid * (POS_PER_W * D)
    for c in range(NCHUNK):
        pos_off = pos_base + c * CHW
        pltpu.sync_copy(pos_hbm.at[pl.ds(pos_off, CHW)], pos_v)
        for b in range(B):
            off = b * (S * D) + pos_off
            pltpu.sync_copy(x_hbm.at[pl.ds(off, CHW)], x_v)

            @plsc.parallel_loop(0, CHW, 16, unroll=8)
            def _(j):
                x_v[pl.ds(j, 16)] = x_v[pl.ds(j, 16)] + pos_v[pl.ds(j, 16)]

            pltpu.sync_copy(x_v, out_hbm.at[pl.ds(off, CHW)])


def kernel(x, pos_table):
    Bx, Sx, Dx = x.shape
    out = _sc_add(x.reshape(-1), pos_table[:Sx].reshape(-1))
    return out.reshape(Bx, Sx, Dx)


# speedup vs baseline: 1.2127x; 1.2127x over previous
import functools

import jax
import jax.numpy as jnp
from jax import lax
from jax.experimental import pallas as pl
from jax.experimental.pallas import tpu as pltpu
from jax.experimental.pallas import tpu_sc as plsc

B, S, D = 4, 8192, 768
NC, NS = 2, 16
NW = NC * NS
POS_PER_W = S // NW
CH = 32
NCHUNK = POS_PER_W // CH
CHW = CH * D
STEPS = NCHUNK * B

_mesh = plsc.VectorSubcoreMesh(core_axis_name="c", subcore_axis_name="s")


@functools.partial(
    pl.kernel,
    out_type=jax.ShapeDtypeStruct((B * S * D,), jnp.float32),
    mesh=_mesh,
    scratch_types=[
        pltpu.VMEM((CHW,), jnp.float32),
        pltpu.VMEM((CHW,), jnp.float32),
        pltpu.VMEM((CHW,), jnp.float32),
        pltpu.VMEM((CHW,), jnp.float32),
        pltpu.SemaphoreType.DMA,
        pltpu.SemaphoreType.DMA,
        pltpu.SemaphoreType.DMA,
        pltpu.SemaphoreType.DMA,
        pltpu.SemaphoreType.DMA,
        pltpu.SemaphoreType.DMA,
    ],
)
def _sc_add(x_hbm, pos_hbm, out_hbm, pv0, pv1, xv0, xv1,
            ps0, ps1, is0, is1, os0, os1):
    wid = lax.axis_index("s") * NC + lax.axis_index("c")
    pos_base = wid * (POS_PER_W * D)
    pbufs, psems = (pv0, pv1), (ps0, ps1)
    xbufs, isems, osems = (xv0, xv1), (is0, is1), (os0, os1)

    def x_off(i):
        c, b = divmod(i, B)
        return b * (S * D) + pos_base + c * CHW

    pos_d = {0: pltpu.async_copy(
        pos_hbm.at[pl.ds(pos_base, CHW)], pv0, ps0)}
    in_d = {0: pltpu.async_copy(
        x_hbm.at[pl.ds(x_off(0), CHW)], xv0, is0)}
    out_d = {}

    for i in range(STEPS):
        c, b = divmod(i, B)
        buf = xbufs[i % 2]
        pbuf = pbufs[c % 2]
        if b == 0:
            pos_d[c].wait()
            if c + 1 < NCHUNK:
                nxt = pos_base + (c + 1) * CHW
                pos_d[c + 1] = pltpu.async_copy(
                    pos_hbm.at[pl.ds(nxt, CHW)], pbufs[(c + 1) % 2],
                    psems[(c + 1) % 2])
        if i + 1 < STEPS:
            if i - 1 >= 0:
                out_d[i - 1].wait()
            in_d[i + 1] = pltpu.async_copy(
                x_hbm.at[pl.ds(x_off(i + 1), CHW)], xbufs[(i + 1) % 2],
                isems[(i + 1) % 2])
        in_d[i].wait()

        @plsc.parallel_loop(0, CHW, 16, unroll=8)
        def _(j):
            plsc.addupdate(buf.at[pl.ds(j, 16)], pbuf[pl.ds(j, 16)])

        out_d[i] = pltpu.async_copy(
            buf, out_hbm.at[pl.ds(x_off(i), CHW)], osems[i % 2])

    out_d[STEPS - 2].wait()
    out_d[STEPS - 1].wait()


def kernel(x, pos_table):
    Bx, Sx, Dx = x.shape
    out = _sc_add(x.reshape(-1), pos_table[:Sx].reshape(-1))
    return out.reshape(Bx, Sx, Dx)


# speedup vs baseline: 3.2947x; 2.7167x over previous
import functools

import jax
import jax.numpy as jnp
from jax import lax
from jax.experimental import pallas as pl
from jax.experimental.pallas import tpu as pltpu
from jax.experimental.pallas import tpu_sc as plsc

B, S, D = 4, 8192, 768
NC, NS = 2, 16
NW = NC * NS
POS_PER_W = S // NW
CH = 32
NCHUNK = POS_PER_W // CH
NVEC = D // 16
STEPS = NCHUNK * B

_mesh = plsc.VectorSubcoreMesh(core_axis_name="c", subcore_axis_name="s")


@functools.partial(
    pl.kernel,
    out_type=jax.ShapeDtypeStruct((B, S, D), jnp.float32),
    mesh=_mesh,
    compiler_params=pltpu.CompilerParams(use_tc_tiling_on_sc=True),
    scratch_types=[
        pltpu.VMEM((CH, D), jnp.float32),
        pltpu.VMEM((CH, D), jnp.float32),
        pltpu.VMEM((CH, D), jnp.float32),
        pltpu.VMEM((CH, D), jnp.float32),
        pltpu.SemaphoreType.DMA,
        pltpu.SemaphoreType.DMA,
        pltpu.SemaphoreType.DMA,
        pltpu.SemaphoreType.DMA,
        pltpu.SemaphoreType.DMA,
        pltpu.SemaphoreType.DMA,
    ],
)
def _sc_add(x_hbm, pos_hbm, out_hbm, pv0, pv1, xv0, xv1,
            ps0, ps1, is0, is1, os0, os1):
    wid = lax.axis_index("s") * NC + lax.axis_index("c")
    row_base = wid * POS_PER_W
    pbufs, psems = (pv0, pv1), (ps0, ps1)
    xbufs, isems, osems = (xv0, xv1), (is0, is1), (os0, os1)

    def rows(i):
        c, b = divmod(i, B)
        return b, row_base + c * CH

    def x_in(i, sem_set=isems):
        b, r0 = rows(i)
        return pltpu.async_copy(
            x_hbm.at[b, pl.ds(r0, CH), :], xbufs[i % 2], sem_set[i % 2])

    pos_d = {0: pltpu.async_copy(
        pos_hbm.at[pl.ds(row_base, CH), :], pv0, ps0)}
    in_d = {0: x_in(0)}
    out_d = {}

    for i in range(STEPS):
        c, b = divmod(i, B)
        buf = xbufs[i % 2]
        pbuf = pbufs[c % 2]
        if b == 0:
            pos_d[c].wait()
            if c + 1 < NCHUNK:
                pos_d[c + 1] = pltpu.async_copy(
                    pos_hbm.at[pl.ds(row_base + (c + 1) * CH, CH), :],
                    pbufs[(c + 1) % 2], psems[(c + 1) % 2])
        if i + 1 < STEPS:
            if i - 1 >= 0:
                out_d[i - 1].wait()
            in_d[i + 1] = x_in(i + 1)
        in_d[i].wait()

        @plsc.parallel_loop(0, CH, 1)
        def _(r):
            @plsc.parallel_loop(0, D, 16, unroll=4)
            def _(j):
                plsc.addupdate(buf.at[r, pl.ds(j, 16)],
                               pbuf[r, pl.ds(j, 16)])

        b_i, r0_i = rows(i)
        out_d[i] = pltpu.async_copy(
            buf, out_hbm.at[b_i, pl.ds(r0_i, CH), :], osems[i % 2])

    out_d[STEPS - 2].wait()
    out_d[STEPS - 1].wait()


def kernel(x, pos_table):
    Bx, Sx, Dx = x.shape
    return _sc_add(x, pos_table[:Sx])


# speedup vs baseline: 3.5429x; 1.0754x over previous
import functools

import jax
import jax.numpy as jnp
from jax import lax
from jax.experimental import pallas as pl
from jax.experimental.pallas import tpu as pltpu
from jax.experimental.pallas import tpu_sc as plsc

B, S, D = 4, 8192, 768
NC, NS = 2, 16
NW = NC * NS
POS_PER_W = S // NW
CH = 32
NCHUNK = POS_PER_W // CH
STEPS = NCHUNK * B
NXB = 3

_mesh = plsc.VectorSubcoreMesh(core_axis_name="c", subcore_axis_name="s")


@functools.partial(
    pl.kernel,
    out_type=jax.ShapeDtypeStruct((B, S, D), jnp.float32),
    mesh=_mesh,
    compiler_params=pltpu.CompilerParams(use_tc_tiling_on_sc=True),
    scratch_types=[
        pltpu.VMEM((CH, D), jnp.float32),
        pltpu.VMEM((CH, D), jnp.float32),
        pltpu.VMEM((CH, D), jnp.float32),
        pltpu.VMEM((CH, D), jnp.float32),
        pltpu.VMEM((CH, D), jnp.float32),
        pltpu.SemaphoreType.DMA,
        pltpu.SemaphoreType.DMA,
        pltpu.SemaphoreType.DMA,
        pltpu.SemaphoreType.DMA,
        pltpu.SemaphoreType.DMA,
        pltpu.SemaphoreType.DMA,
        pltpu.SemaphoreType.DMA,
        pltpu.SemaphoreType.DMA,
    ],
)
def _sc_add(x_hbm, pos_hbm, out_hbm, pv0, pv1, xv0, xv1, xv2,
            ps0, ps1, is0, is1, is2, os0, os1, os2):
    wid = lax.axis_index("s") * NC + lax.axis_index("c")
    row_base = wid * POS_PER_W
    pbufs, psems = (pv0, pv1), (ps0, ps1)
    xbufs, isems, osems = (xv0, xv1, xv2), (is0, is1, is2), (os0, os1, os2)

    def rows(i):
        c, b = divmod(i, B)
        return b, row_base + c * CH

    def x_in(i):
        b, r0 = rows(i)
        return pltpu.async_copy(
            x_hbm.at[b, pl.ds(r0, CH), :], xbufs[i % NXB], isems[i % NXB])

    pos_d = {0: pltpu.async_copy(
        pos_hbm.at[pl.ds(row_base, CH), :], pv0, ps0)}
    in_d = {0: x_in(0), 1: x_in(1)}
    out_d = {}

    for i in range(STEPS):
        c, b = divmod(i, B)
        buf = xbufs[i % NXB]
        pbuf = pbufs[c % 2]
        if b == 0:
            pos_d[c].wait()
            if c + 1 < NCHUNK:
                pos_d[c + 1] = pltpu.async_copy(
                    pos_hbm.at[pl.ds(row_base + (c + 1) * CH, CH), :],
                    pbufs[(c + 1) % 2], psems[(c + 1) % 2])
        if i + 2 < STEPS:
            if i - 1 >= 0:
                out_d[i - 1].wait()
            in_d[i + 2] = x_in(i + 2)
        in_d[i].wait()

        @plsc.parallel_loop(0, CH, 1)
        def _(r):
            @plsc.parallel_loop(0, D, 16, unroll=8)
            def _(j):
                plsc.addupdate(buf.at[r, pl.ds(j, 16)],
                               pbuf[r, pl.ds(j, 16)])

        b_i, r0_i = rows(i)
        out_d[i] = pltpu.async_copy(
            buf, out_hbm.at[b_i, pl.ds(r0_i, CH), :], osems[i % NXB])

    out_d[STEPS - 2].wait()
    out_d[STEPS - 1].wait()


def kernel(x, pos_table):
    Bx, Sx, Dx = x.shape
    return _sc_add(x, pos_table[:Sx])


# speedup vs baseline: 3.5485x; 1.0016x over previous
import functools

import jax
import jax.numpy as jnp
from jax import lax
from jax.experimental import pallas as pl
from jax.experimental.pallas import tpu as pltpu
from jax.experimental.pallas import tpu_sc as plsc

B, S, D = 4, 8192, 768
NC, NS = 2, 16
NW = NC * NS
POS_PER_W = S // NW
CH = 32
NCHUNK = POS_PER_W // CH
STEPS = NCHUNK * B
NXB = 3

_mesh = plsc.VectorSubcoreMesh(core_axis_name="c", subcore_axis_name="s")


@functools.partial(
    pl.kernel,
    out_type=jax.ShapeDtypeStruct((B, S, D), jnp.float32),
    mesh=_mesh,
    compiler_params=pltpu.CompilerParams(use_tc_tiling_on_sc=True),
    scratch_types=[
        pltpu.VMEM((CH, D), jnp.float32),
        pltpu.VMEM((CH, D), jnp.float32),
        pltpu.VMEM((CH, D), jnp.float32),
        pltpu.VMEM((CH, D), jnp.float32),
        pltpu.VMEM((CH, D), jnp.float32),
        pltpu.SemaphoreType.DMA,
        pltpu.SemaphoreType.DMA,
        pltpu.SemaphoreType.DMA,
        pltpu.SemaphoreType.DMA,
        pltpu.SemaphoreType.DMA,
        pltpu.SemaphoreType.DMA,
        pltpu.SemaphoreType.DMA,
        pltpu.SemaphoreType.DMA,
    ],
)
def _sc_add(x_hbm, pos_hbm, out_hbm, pv0, pv1, xv0, xv1, xv2,
            ps0, ps1, is0, is1, is2, os0, os1, os2):
    wid = lax.axis_index("s") * NC + lax.axis_index("c")
    row_base = wid * POS_PER_W
    pbufs, psems = (pv0, pv1), (ps0, ps1)
    xbufs, isems, osems = (xv0, xv1, xv2), (is0, is1, is2), (os0, os1, os2)

    def rows(i):
        c, b = divmod(i, B)
        return b, row_base + c * CH

    def x_in(i):
        b, r0 = rows(i)
        return pltpu.async_copy(
            x_hbm.at[b, pl.ds(r0, CH), :], xbufs[i % NXB], isems[i % NXB])

    pos_d = {0: pltpu.async_copy(
        pos_hbm.at[pl.ds(row_base, CH), :], pv0, ps0)}
    in_d = {0: x_in(0), 1: x_in(1)}
    out_d = {}

    for i in range(STEPS):
        c, b = divmod(i, B)
        buf = xbufs[i % NXB]
        pbuf = pbufs[c % 2]
        if b == 0:
            pos_d[c].wait()
            if c + 1 < NCHUNK:
                pos_d[c + 1] = pltpu.async_copy(
                    pos_hbm.at[pl.ds(row_base + (c + 1) * CH, CH), :],
                    pbufs[(c + 1) % 2], psems[(c + 1) % 2])
        if i + 2 < STEPS:
            if i - 1 >= 0:
                out_d[i - 1].wait()
            in_d[i + 2] = x_in(i + 2)
        in_d[i].wait()

        @plsc.parallel_loop(0, CH, 1, unroll=2)
        def _(r):
            @plsc.parallel_loop(0, D, 16, unroll=16)
            def _(j):
                plsc.addupdate(buf.at[r, pl.ds(j, 16)],
                               pbuf[r, pl.ds(j, 16)])

        b_i, r0_i = rows(i)
        out_d[i] = pltpu.async_copy(
            buf, out_hbm.at[b_i, pl.ds(r0_i, CH), :], osems[i % NXB])

    out_d[STEPS - 2].wait()
    out_d[STEPS - 1].wait()


def kernel(x, pos_table):
    Bx, Sx, Dx = x.shape
    return _sc_add(x, pos_table[:Sx])


# speedup vs baseline: 3.7834x; 1.0662x over previous
import functools

import jax
import jax.numpy as jnp
from jax import lax
from jax.experimental import pallas as pl
from jax.experimental.pallas import tpu as pltpu
from jax.experimental.pallas import tpu_sc as plsc

B, S, D = 4, 8192, 768
NC, NS = 2, 16
NW = NC * NS
POS_PER_W = S // NW
CH = 16
NCHUNK = POS_PER_W // CH
STEPS = NCHUNK * B
NXB = 6

_mesh = plsc.VectorSubcoreMesh(core_axis_name="c", subcore_axis_name="s")


@functools.partial(
    pl.kernel,
    out_type=jax.ShapeDtypeStruct((B, S, D), jnp.float32),
    mesh=_mesh,
    compiler_params=pltpu.CompilerParams(use_tc_tiling_on_sc=True),
    scratch_types=[
        pltpu.VMEM((CH, D), jnp.float32),
        pltpu.VMEM((CH, D), jnp.float32),
        pltpu.VMEM((CH, D), jnp.float32),
        pltpu.VMEM((CH, D), jnp.float32),
        pltpu.VMEM((CH, D), jnp.float32),
        pltpu.VMEM((CH, D), jnp.float32),
        pltpu.VMEM((CH, D), jnp.float32),
        pltpu.VMEM((CH, D), jnp.float32),
        pltpu.SemaphoreType.DMA,
        pltpu.SemaphoreType.DMA,
        pltpu.SemaphoreType.DMA,
        pltpu.SemaphoreType.DMA,
        pltpu.SemaphoreType.DMA,
        pltpu.SemaphoreType.DMA,
        pltpu.SemaphoreType.DMA,
        pltpu.SemaphoreType.DMA,
        pltpu.SemaphoreType.DMA,
        pltpu.SemaphoreType.DMA,
        pltpu.SemaphoreType.DMA,
        pltpu.SemaphoreType.DMA,
        pltpu.SemaphoreType.DMA,
        pltpu.SemaphoreType.DMA,
    ],
)
def _sc_add(x_hbm, pos_hbm, out_hbm, pv0, pv1,
            xv0, xv1, xv2, xv3, xv4, xv5,
            ps0, ps1, is0, is1, is2, is3, is4, is5,
            os0, os1, os2, os3, os4, os5):
    wid = lax.axis_index("s") * NC + lax.axis_index("c")
    row_base = wid * POS_PER_W
    pbufs, psems = (pv0, pv1), (ps0, ps1)
    xbufs = (xv0, xv1, xv2, xv3, xv4, xv5)
    isems = (is0, is1, is2, is3, is4, is5)
    osems = (os0, os1, os2, os3, os4, os5)

    def rows(i):
        c, b = divmod(i, B)
        return b, row_base + c * CH

    def x_in(i):
        b, r0 = rows(i)
        return pltpu.async_copy(
            x_hbm.at[b, pl.ds(r0, CH), :], xbufs[i % NXB], isems[i % NXB])

    pos_d = {0: pltpu.async_copy(
        pos_hbm.at[pl.ds(row_base, CH), :], pv0, ps0)}
    in_d = {i: x_in(i) for i in range(4)}
    out_d = {}

    for i in range(STEPS):
        c, b = divmod(i, B)
        buf = xbufs[i % NXB]
        pbuf = pbufs[c % 2]
        if b == 0:
            pos_d[c].wait()
            if c + 1 < NCHUNK:
                pos_d[c + 1] = pltpu.async_copy(
                    pos_hbm.at[pl.ds(row_base + (c + 1) * CH, CH), :],
                    pbufs[(c + 1) % 2], psems[(c + 1) % 2])
        if i + 4 < STEPS:
            if i - 2 >= 0:
                out_d[i - 2].wait()
            in_d[i + 4] = x_in(i + 4)
        in_d[i].wait()

        @plsc.parallel_loop(0, CH, 1)
        def _(r):
            @plsc.parallel_loop(0, D, 16, unroll=8)
            def _(j):
                plsc.addupdate(buf.at[r, pl.ds(j, 16)],
                               pbuf[r, pl.ds(j, 16)])

        b_i, r0_i = rows(i)
        out_d[i] = pltpu.async_copy(
            buf, out_hbm.at[b_i, pl.ds(r0_i, CH), :], osems[i % NXB])

    for k in range(4):
        out_d[STEPS - 4 + k].wait()


def kernel(x, pos_table):
    Bx, Sx, Dx = x.shape
    return _sc_add(x, pos_table[:Sx])
